# baseline (speedup 1.0000x reference)
import jax
import jax.numpy as jnp
from jax.experimental import pallas as pl
from jax.experimental.pallas import tpu as pltpu

_WALL_COT = 0.5
_NSEG = 8


def _loss_body(out_ref, in_ref, m_ref, s_ref, e_ref, d_ref,
               loss_ref, cnt_acc, pos_acc, err_acc, recov_acc):
    b = pl.program_id(0)
    hc = pl.program_id(1)
    nb = pl.num_programs(0)
    nhc = pl.num_programs(1)

    @pl.when(jnp.logical_and(b == 0, hc == 0))
    def _init():
        cnt_acc[...] = jnp.zeros_like(cnt_acc)
        pos_acc[...] = jnp.zeros_like(pos_acc)
        err_acc[...] = jnp.zeros_like(err_acc)
        recov_acc[...] = jnp.zeros_like(recov_acc)

    m = m_ref[0, 0]
    o = out_ref[0]
    x = in_ref[0]
    t = jnp.where(m[None] >= _WALL_COT, 0.0, x)
    diff = o - t
    mse = jnp.sum(diff * diff, axis=0)
    mpos = m > 0.0
    recov_sum = jnp.sum(jnp.where(mpos, mse, 0.0), axis=0)
    recov_cnt = jnp.sum(mpos.astype(jnp.float32), axis=0)
    recov_acc[0:1, :] = recov_acc[0:1, :] + recov_sum[None]
    recov_acc[1:2, :] = recov_acc[1:2, :] + recov_cnt[None]

    e = e_ref[0]
    d = d_ref[0]
    ed = e - d
    re = jnp.sum(ed * ed, axis=0) / 96.0

    echunk, wechunk = e_ref.shape[2], e_ref.shape[3]
    hchunk, wchunk = m_ref.shape[2], m_ref.shape[3]
    he_i = jax.lax.broadcasted_iota(jnp.int32, (echunk, hchunk), 0)
    h_i = jax.lax.broadcasted_iota(jnp.int32, (echunk, hchunk), 1)
    p2 = (h_i == 4 * he_i).astype(jnp.float32)
    w_i = jax.lax.broadcasted_iota(jnp.int32, (wchunk, wechunk), 0)
    we_i = jax.lax.broadcasted_iota(jnp.int32, (wchunk, wechunk), 1)
    p1 = (w_i == 4 * we_i).astype(jnp.float32)

    seg = s_ref[0, :, 0, 0]
    pm = jnp.logical_and(m < _WALL_COT, m > 0.0).astype(jnp.float32)
    seg_sub = jnp.dot(seg, p1, preferred_element_type=jnp.float32)
    pm_sub = jnp.dot(jnp.dot(p2, pm, preferred_element_type=jnp.float32),
                     p1, preferred_element_type=jnp.float32)

    cnt_rows = []
    pos_rows = []
    err_rows = []
    for s in range(_NSEG):
        ms = (seg_sub == float(s)).astype(jnp.float32)
        cnt_rows.append(jnp.sum(ms, axis=0)[None])
        pos_rows.append(jnp.sum(ms * pm_sub, axis=0)[None])
        err_rows.append(jnp.sum(ms * re, axis=0)[None])
    rows = pl.ds(b * _NSEG, _NSEG)
    cnt_acc[rows, :] = cnt_acc[rows, :] + jnp.concatenate(cnt_rows, axis=0)
    pos_acc[rows, :] = pos_acc[rows, :] + jnp.concatenate(pos_rows, axis=0)
    err_acc[rows, :] = err_acc[rows, :] + jnp.concatenate(err_rows, axis=0)

    @pl.when(jnp.logical_and(b == nb - 1, hc == nhc - 1))
    def _finish():
        cnt = jnp.sum(cnt_acc[...], axis=1, keepdims=True)
        pos = jnp.sum(pos_acc[...], axis=1, keepdims=True)
        err = jnp.sum(err_acc[...], axis=1, keepdims=True)
        valid = jnp.logical_not(cnt / 16384.0 < 0.01)
        mean_err = err / cnt
        flags = jnp.logical_and(valid, pos / cnt > 0.01)
        pos_sum = jnp.sum(jnp.where(flags, mean_err, 0.0))
        pos_cnt = jnp.sum(flags.astype(jnp.float32))
        rs = jnp.sum(recov_acc[0:1, :])
        rc = jnp.sum(recov_acc[1:2, :])
        loss = rs / rc + pos_sum / pos_cnt
        loss_ref[...] = jnp.broadcast_to(loss, loss_ref.shape)


def kernel(outputs, inputs, enc1, dec1, masks, segs, confidence,
           iteration, epoch):
    B, C, H, W = outputs.shape
    _, Ce, He, We = enc1.shape
    nhc = 4
    hchunk = H // nhc
    echunk = He // nhc

    grid = (B, nhc)
    loss_out = pl.pallas_call(
        _loss_body,
        grid=grid,
        in_specs=[
            pl.BlockSpec((1, C, hchunk, W), lambda b, h: (b, 0, h, 0)),
            pl.BlockSpec((1, C, hchunk, W), lambda b, h: (b, 0, h, 0)),
            pl.BlockSpec((1, 1, hchunk, W), lambda b, h: (b, 0, h, 0)),
            pl.BlockSpec((1, echunk, 1, 1, W), lambda b, h: (b, h, 0, 0, 0)),
            pl.BlockSpec((1, Ce, echunk, We), lambda b, h: (b, 0, h, 0)),
            pl.BlockSpec((1, Ce, echunk, We), lambda b, h: (b, 0, h, 0)),
        ],
        out_specs=pl.BlockSpec((8, 128), lambda b, h: (0, 0)),
        out_shape=jax.ShapeDtypeStruct((8, 128), jnp.float32),
        scratch_shapes=[
            pltpu.VMEM((B * _NSEG, We), jnp.float32),
            pltpu.VMEM((B * _NSEG, We), jnp.float32),
            pltpu.VMEM((B * _NSEG, We), jnp.float32),
            pltpu.VMEM((8, W), jnp.float32),
        ],
        compiler_params=pltpu.CompilerParams(
            dimension_semantics=("arbitrary", "arbitrary")),
    )(outputs, inputs, masks,
      segs.reshape(B, He, H // He, 1, W), enc1, dec1)
    return loss_out[0, 0]


# speedup vs baseline: 1.0224x; 1.0224x over previous
import functools
import jax
import jax.numpy as jnp
from jax import lax
from jax.experimental import pallas as pl
from jax.experimental.pallas import tpu as pltpu
from jax.experimental.pallas import tpu_sc as plsc

_WALL_COT = 0.5
_NSEG = 8


def _dense_body(out_ref, in_ref, m_ref, s_ref, e_ref, d_ref,
                recov_o, re_o, segsub_o, pmsub_o):
    b = pl.program_id(0)
    hc = pl.program_id(1)

    @pl.when(jnp.logical_and(b == 0, hc == 0))
    def _init():
        recov_o[...] = jnp.zeros_like(recov_o)

    m = m_ref[0, 0]
    o = out_ref[0]
    x = in_ref[0]
    t = jnp.where(m[None] >= _WALL_COT, 0.0, x)
    diff = o - t
    mse = jnp.sum(diff * diff, axis=0)
    mpos = m > 0.0
    recov_sum = jnp.sum(jnp.where(mpos, mse, 0.0), axis=0)
    recov_cnt = jnp.sum(mpos.astype(jnp.float32), axis=0)
    recov_o[0:1, :] = recov_o[0:1, :] + recov_sum[None]
    recov_o[1:2, :] = recov_o[1:2, :] + recov_cnt[None]

    e = e_ref[0]
    d = d_ref[0]
    ed = e - d
    re_o[0] = jnp.sum(ed * ed, axis=0) / 96.0

    echunk, hchunk = e_ref.shape[2], m_ref.shape[2]
    wchunk, wechunk = m_ref.shape[3], e_ref.shape[3]
    he_i = jax.lax.broadcasted_iota(jnp.int32, (echunk, hchunk), 0)
    h_i = jax.lax.broadcasted_iota(jnp.int32, (echunk, hchunk), 1)
    p2 = (h_i == 4 * he_i).astype(jnp.float32)
    w_i = jax.lax.broadcasted_iota(jnp.int32, (wchunk, wechunk), 0)
    we_i = jax.lax.broadcasted_iota(jnp.int32, (wchunk, wechunk), 1)
    p1 = (w_i == 4 * we_i).astype(jnp.float32)
    seg = s_ref[0, 0]
    pm = jnp.logical_and(m < _WALL_COT, m > 0.0).astype(jnp.float32)
    segsub_o[0] = jnp.dot(jnp.dot(p2, seg, preferred_element_type=jnp.float32),
                          p1, preferred_element_type=jnp.float32)
    pmsub_o[0] = jnp.dot(jnp.dot(p2, pm, preferred_element_type=jnp.float32),
                         p1, preferred_element_type=jnp.float32)


def _sc_stats(seg_hbm, pm_hbm, re_hbm, out_hbm,
              seg1, pm1, re1, acc_v, sem0, sem1):
    cid = lax.axis_index("c")
    sid = lax.axis_index("s")
    wid = sid * 2 + cid
    npts = 4096
    base = wid * npts

    cp0 = pltpu.async_copy(seg_hbm.at[pl.ds(base, npts)], seg1, sem0)
    cp1 = pltpu.async_copy(pm_hbm.at[pl.ds(base, npts)], pm1, sem1)
    pltpu.sync_copy(re_hbm.at[pl.ds(base, npts)], re1)
    cp0.wait()
    cp1.wait()

    zero = jnp.zeros((16,), jnp.float32)
    carry = tuple(zero for _ in range(3 * _NSEG))

    def body(r, acc):
        acc = list(acc)
        segv = seg1[pl.ds(r * 16, 16)]
        pmv = pm1[pl.ds(r * 16, 16)]
        rev = re1[pl.ds(r * 16, 16)]
        for s in range(_NSEG):
            msk = segv == float(s)
            acc[s] = acc[s] + jnp.where(msk, 1.0, 0.0)
            acc[_NSEG + s] = acc[_NSEG + s] + jnp.where(msk, pmv, 0.0)
            acc[2 * _NSEG + s] = acc[2 * _NSEG + s] + jnp.where(
                msk, rev, 0.0)
        return tuple(acc)

    acc = lax.fori_loop(0, 256, body, carry)
    for i in range(3 * _NSEG):
        acc_v[i, :] = acc[i]
    pltpu.sync_copy(acc_v, out_hbm.at[wid])


def _combine_body(sc_ref, recov_ref, loss_ref):
    v = jnp.sum(sc_ref[...], axis=2)
    v4 = v.reshape(_NSEG, 4, 3 * _NSEG)
    g = jnp.sum(v4, axis=1)
    cnt = g[:, 0:_NSEG]
    pos = g[:, _NSEG:2 * _NSEG]
    err = g[:, 2 * _NSEG:3 * _NSEG]
    valid = jnp.logical_not(cnt / 16384.0 < 0.01)
    mean_err = err / cnt
    flags = jnp.logical_and(valid, pos / cnt > 0.01)
    pos_sum = jnp.sum(jnp.where(flags, mean_err, 0.0))
    pos_cnt = jnp.sum(flags.astype(jnp.float32))
    rs = jnp.sum(recov_ref[0:1, :])
    rc = jnp.sum(recov_ref[1:2, :])
    loss = rs / rc + pos_sum / pos_cnt
    loss_ref[...] = jnp.broadcast_to(loss, loss_ref.shape)


def kernel(outputs, inputs, enc1, dec1, masks, segs, confidence,
           iteration, epoch):
    B, C, H, W = outputs.shape
    _, Ce, He, We = enc1.shape
    nhc = 4
    hchunk = H // nhc
    echunk = He // nhc
    f32 = jnp.float32

    recov_o, re_o, segsub_o, pmsub_o = pl.pallas_call(
        _dense_body,
        grid=(B, nhc),
        in_specs=[
            pl.BlockSpec((1, C, hchunk, W), lambda b, h: (b, 0, h, 0)),
            pl.BlockSpec((1, C, hchunk, W), lambda b, h: (b, 0, h, 0)),
            pl.BlockSpec((1, 1, hchunk, W), lambda b, h: (b, 0, h, 0)),
            pl.BlockSpec((1, 1, hchunk, W), lambda b, h: (b, 0, h, 0)),
            pl.BlockSpec((1, Ce, echunk, We), lambda b, h: (b, 0, h, 0)),
            pl.BlockSpec((1, Ce, echunk, We), lambda b, h: (b, 0, h, 0)),
        ],
        out_specs=[
            pl.BlockSpec((8, W), lambda b, h: (0, 0)),
            pl.BlockSpec((1, echunk, We), lambda b, h: (b, h, 0)),
            pl.BlockSpec((1, echunk, We), lambda b, h: (b, h, 0)),
            pl.BlockSpec((1, echunk, We), lambda b, h: (b, h, 0)),
        ],
        out_shape=[
            jax.ShapeDtypeStruct((8, W), f32),
            jax.ShapeDtypeStruct((B, He, We), f32),
            jax.ShapeDtypeStruct((B, He, We), f32),
            jax.ShapeDtypeStruct((B, He, We), f32),
        ],
        compiler_params=pltpu.CompilerParams(
            dimension_semantics=("arbitrary", "arbitrary")),
    )(outputs, inputs, masks, segs, enc1, dec1)

    mesh = plsc.VectorSubcoreMesh(core_axis_name="c", subcore_axis_name="s")
    sc_stats = functools.partial(
        pl.kernel, mesh=mesh,
        out_type=jax.ShapeDtypeStruct((32, 3 * _NSEG, 16), f32),
        scratch_types=[
            pltpu.VMEM((4096,), f32),
            pltpu.VMEM((4096,), f32),
            pltpu.VMEM((4096,), f32),
            pltpu.VMEM((3 * _NSEG, 16), f32),
            pltpu.SemaphoreType.DMA,
            pltpu.SemaphoreType.DMA,
        ],
        compiler_params=pltpu.CompilerParams(use_tc_tiling_on_sc=False),
    )(_sc_stats)
    sc_out = sc_stats(segsub_o.reshape(B * He * We),
                      pmsub_o.reshape(B * He * We),
                      re_o.reshape(B * He * We))

    loss_out = pl.pallas_call(
        _combine_body,
        out_shape=jax.ShapeDtypeStruct((8, 128), f32),
    )(sc_out, recov_o)
    return loss_out[0, 0]


# speedup vs baseline: 1.4371x; 1.4056x over previous
import jax
import jax.numpy as jnp
from jax.experimental import pallas as pl
from jax.experimental.pallas import tpu as pltpu

_WALL_COT = 0.5
_NSEG = 8


def _loss_body(out_ref, in_ref, m_ref, s_ref, e_ref, d_ref,
               loss_ref, cnt_acc, pos_acc, err_acc, recov_acc):
    b = pl.program_id(0)
    hc = pl.program_id(1)
    nb = pl.num_programs(0)
    nhc = pl.num_programs(1)

    @pl.when(jnp.logical_and(b == 0, hc == 0))
    def _init():
        cnt_acc[...] = jnp.zeros_like(cnt_acc)
        pos_acc[...] = jnp.zeros_like(pos_acc)
        err_acc[...] = jnp.zeros_like(err_acc)
        recov_acc[...] = jnp.zeros_like(recov_acc)

    m = m_ref[0, 0]
    o = out_ref[0]
    x = in_ref[0]
    t = jnp.where(m[None] >= _WALL_COT, 0.0, x)
    diff = o - t
    mse = jnp.sum(diff * diff, axis=0)
    mpos = m > 0.0
    recov_sum = jnp.sum(jnp.where(mpos, mse, 0.0), axis=0)
    recov_cnt = jnp.sum(mpos.astype(jnp.float32), axis=0)
    recov_acc[0:1, :] = recov_acc[0:1, :] + recov_sum[None]
    recov_acc[1:2, :] = recov_acc[1:2, :] + recov_cnt[None]

    e = e_ref[0]
    d = d_ref[0]
    ed = e - d
    re = jnp.sum(ed * ed, axis=0) / 96.0

    echunk, wechunk = e_ref.shape[2], e_ref.shape[3]
    hchunk, wchunk = m_ref.shape[2], m_ref.shape[3]
    he_i = jax.lax.broadcasted_iota(jnp.int32, (echunk, hchunk), 0)
    h_i = jax.lax.broadcasted_iota(jnp.int32, (echunk, hchunk), 1)
    p2 = (h_i == 4 * he_i).astype(jnp.float32)
    w_i = jax.lax.broadcasted_iota(jnp.int32, (wchunk, wechunk), 0)
    we_i = jax.lax.broadcasted_iota(jnp.int32, (wchunk, wechunk), 1)
    p1 = (w_i == 4 * we_i).astype(jnp.float32)

    seg = s_ref[0, 0]
    pm = jnp.logical_and(m < _WALL_COT, m > 0.0).astype(jnp.float32)
    seg_sub = jnp.dot(jnp.dot(p2, seg, preferred_element_type=jnp.float32),
                      p1, preferred_element_type=jnp.float32)
    pm_sub = jnp.dot(jnp.dot(p2, pm, preferred_element_type=jnp.float32),
                     p1, preferred_element_type=jnp.float32)

    cnt_rows = []
    pos_rows = []
    err_rows = []
    for s in range(_NSEG):
        ms = (seg_sub == float(s)).astype(jnp.float32)
        cnt_rows.append(jnp.sum(ms, axis=0)[None])
        pos_rows.append(jnp.sum(ms * pm_sub, axis=0)[None])
        err_rows.append(jnp.sum(ms * re, axis=0)[None])
    rows = pl.ds(b * _NSEG, _NSEG)
    cnt_acc[rows, :] = cnt_acc[rows, :] + jnp.concatenate(cnt_rows, axis=0)
    pos_acc[rows, :] = pos_acc[rows, :] + jnp.concatenate(pos_rows, axis=0)
    err_acc[rows, :] = err_acc[rows, :] + jnp.concatenate(err_rows, axis=0)

    @pl.when(jnp.logical_and(b == nb - 1, hc == nhc - 1))
    def _finish():
        cnt = jnp.sum(cnt_acc[...], axis=1, keepdims=True)
        pos = jnp.sum(pos_acc[...], axis=1, keepdims=True)
        err = jnp.sum(err_acc[...], axis=1, keepdims=True)
        valid = jnp.logical_not(cnt / 16384.0 < 0.01)
        mean_err = err / cnt
        flags = jnp.logical_and(valid, pos / cnt > 0.01)
        pos_sum = jnp.sum(jnp.where(flags, mean_err, 0.0))
        pos_cnt = jnp.sum(flags.astype(jnp.float32))
        rs = jnp.sum(recov_acc[0:1, :])
        rc = jnp.sum(recov_acc[1:2, :])
        loss = rs / rc + pos_sum / pos_cnt
        loss_ref[...] = jnp.broadcast_to(loss, loss_ref.shape)


def kernel(outputs, inputs, enc1, dec1, masks, segs, confidence,
           iteration, epoch):
    B, C, H, W = outputs.shape
    _, Ce, He, We = enc1.shape
    nhc = 4
    hchunk = H // nhc
    echunk = He // nhc

    grid = (B, nhc)
    loss_out = pl.pallas_call(
        _loss_body,
        grid=grid,
        in_specs=[
            pl.BlockSpec((1, C, hchunk, W), lambda b, h: (b, 0, h, 0)),
            pl.BlockSpec((1, C, hchunk, W), lambda b, h: (b, 0, h, 0)),
            pl.BlockSpec((1, 1, hchunk, W), lambda b, h: (b, 0, h, 0)),
            pl.BlockSpec((1, 1, hchunk, W), lambda b, h: (b, 0, h, 0)),
            pl.BlockSpec((1, Ce, echunk, We), lambda b, h: (b, 0, h, 0)),
            pl.BlockSpec((1, Ce, echunk, We), lambda b, h: (b, 0, h, 0)),
        ],
        out_specs=pl.BlockSpec((8, 128), lambda b, h: (0, 0)),
        out_shape=jax.ShapeDtypeStruct((8, 128), jnp.float32),
        scratch_shapes=[
            pltpu.VMEM((B * _NSEG, We), jnp.float32),
            pltpu.VMEM((B * _NSEG, We), jnp.float32),
            pltpu.VMEM((B * _NSEG, We), jnp.float32),
            pltpu.VMEM((8, W), jnp.float32),
        ],
        compiler_params=pltpu.CompilerParams(
            dimension_semantics=("arbitrary", "arbitrary")),
    )(outputs, inputs, masks, segs, enc1, dec1)
    return loss_out[0, 0]
